# baseline (speedup 1.0000x reference)
import functools

import jax
import jax.numpy as jnp
from jax import lax
from jax.experimental import pallas as pl
from jax.experimental.pallas import tpu as pltpu
from jax.experimental.pallas import tpu_sc as plsc

NC = 2
NS = 16
NW = NC * NS
L = 16

IN_F = 4096
OUT_F = 4096
BATCH = 1024
NPB = BATCH // L
NPASS = NPB // NW
K = 16
GB = 8


def _sc_body(nnz_pad, inp_hbm, m0_hbm, m1_hbm, w_hbm, out_hbm,
             m0st, m1st, wst, gbuf, acc, sem_g):
    nbatch = nnz_pad // (GB * K)
    cid = lax.axis_index("c")
    sid = lax.axis_index("s")
    wid = cid * NS + sid

    zeros = jnp.zeros((L,), jnp.float32)

    pltpu.sync_copy(m0_hbm, m0st)
    pltpu.sync_copy(m1_hbm, m1st)
    pltpu.sync_copy(w_hbm, wst)

    def _run_pass(p, carry):
        cb = p * NW + wid
        roff = cb * IN_F

        def _zfill(r8, c2):
            for i in range(8):
                acc[r8 * 8 + i, :] = zeros
            return c2
        lax.fori_loop(0, OUT_F // 8, _zfill, 0)

        def _batch(gb, c2):
            eb = gb * (GB * K)
            cps = []
            for blk in range(GB):
                m0v = m0st[pl.ds(eb + blk * K, K)] + roff
                cps.append(pltpu.async_copy(
                    inp_hbm.at[m0v], gbuf.at[pl.ds(blk * K, K)], sem_g))
            for cp in cps:
                cp.wait()
            for blk in range(GB):
                base = eb + blk * K
                rv = m1st[pl.ds(base, K)]
                wv = wst[pl.ds(base, K)]
                for k in range(K):
                    r = rv[k]
                    g = blk * K + k
                    acc[r, :] = acc[r, :] + wv[k] * gbuf[g, :]
            return c2
        lax.fori_loop(0, nbatch, _batch, 0)

        pltpu.sync_copy(acc, out_hbm.at[pl.ds(cb * OUT_F, OUT_F)])
        return carry

    lax.fori_loop(0, NPASS, _run_pass, 0)


@functools.partial(jax.jit, static_argnames=("nnz_pad",))
def _sc_call(inputX, m0, m1, w, *, nnz_pad):
    mesh = plsc.VectorSubcoreMesh(core_axis_name="c", subcore_axis_name="s")
    return pl.kernel(
        functools.partial(_sc_body, nnz_pad),
        out_type=jax.ShapeDtypeStruct((NPB * OUT_F, L), jnp.float32),
        mesh=mesh,
        compiler_params=pltpu.CompilerParams(use_tc_tiling_on_sc=False),
        scratch_types=[
            pltpu.VMEM((nnz_pad,), jnp.int32),
            pltpu.VMEM((nnz_pad,), jnp.int32),
            pltpu.VMEM((nnz_pad,), jnp.float32),
            pltpu.VMEM((GB * K, L), jnp.float32),
            pltpu.VMEM((OUT_F, L), jnp.float32),
            pltpu.SemaphoreType.DMA,
        ],
    )(inputX, m0, m1, w)


def kernel(input, sparse_mask, weight, bias):
    assert input.shape == (BATCH, IN_F)
    nnz = sparse_mask.shape[0]
    blk = GB * K
    nnz_pad = ((nnz + blk - 1) // blk) * blk
    pad = nnz_pad - nnz

    inputX = (input.reshape(NPB, L, IN_F)
              .transpose(0, 2, 1)
              .reshape(NPB * IN_F, L))
    m0 = jnp.concatenate([sparse_mask[:, 0], jnp.zeros((pad,), jnp.int32)])
    m1 = jnp.concatenate([sparse_mask[:, 1], jnp.zeros((pad,), jnp.int32)])
    w = jnp.concatenate([weight, jnp.zeros((pad,), weight.dtype)])

    outX = _sc_call(inputX, m0, m1, w, nnz_pad=nnz_pad)
    out = (outX.reshape(NPB, OUT_F, L)
           .transpose(0, 2, 1)
           .reshape(BATCH, OUT_F))
    return out + bias[None, :]


# speedup vs baseline: 1.3722x; 1.3722x over previous
import functools

import jax
import jax.numpy as jnp
from jax import lax
from jax.experimental import pallas as pl
from jax.experimental.pallas import tpu as pltpu
from jax.experimental.pallas import tpu_sc as plsc

NC = 2
NS = 16
NW = NC * NS
L = 16

IN_F = 4096
OUT_F = 4096
BATCH = 1024
NPB = BATCH // L
NPASS = NPB // NW
K = 16
GB = 8


def _sc_body(nnz_pad, inp_hbm, m0_hbm, m1_hbm, w_hbm, out_hbm,
             m0st, m1st, wst, gbuf, acc, sems):
    nbatch = nnz_pad // (GB * K)
    cid = lax.axis_index("c")
    sid = lax.axis_index("s")
    wid = cid * NS + sid

    zeros = jnp.zeros((L,), jnp.float32)

    pltpu.sync_copy(m0_hbm, m0st)
    pltpu.sync_copy(m1_hbm, m1st)
    pltpu.sync_copy(w_hbm, wst)

    def _run_pass(p, carry):
        cb = p * NW + wid
        roff = cb * IN_F

        def _zfill(r8, c2):
            for i in range(8):
                acc[r8 * 8 + i, :] = zeros
            return c2
        lax.fori_loop(0, OUT_F // 8, _zfill, 0)

        def _fire(gb):
            eb = gb * (GB * K)
            par = gb % 2
            for blk in range(GB):
                m0v = m0st[pl.ds(eb + blk * K, K)] + roff
                pltpu.async_copy(
                    inp_hbm.at[m0v], gbuf.at[par, pl.ds(blk * K, K)],
                    sems.at[par])

        _fire(0)

        def _batch(gb, c2):
            par = gb % 2

            @pl.when(gb + 1 < nbatch)
            def _prefetch():
                _fire(gb + 1)

            pltpu.make_async_copy(
                inp_hbm.at[pl.ds(0, GB * K)], gbuf.at[par],
                sems.at[par]).wait()

            eb = gb * (GB * K)
            for blk in range(GB):
                base = eb + blk * K
                rv = m1st[pl.ds(base, K)]
                wv = wst[pl.ds(base, K)]
                for k in range(K):
                    r = rv[k]
                    g = blk * K + k
                    acc[r, :] = acc[r, :] + wv[k] * gbuf[par, g, :]
            return c2
        lax.fori_loop(0, nbatch, _batch, 0)

        pltpu.sync_copy(acc, out_hbm.at[pl.ds(cb * OUT_F, OUT_F)])
        return carry

    lax.fori_loop(0, NPASS, _run_pass, 0)


@functools.partial(jax.jit, static_argnames=("nnz_pad",))
def _sc_call(inputX, m0, m1, w, *, nnz_pad):
    mesh = plsc.VectorSubcoreMesh(core_axis_name="c", subcore_axis_name="s")
    return pl.kernel(
        functools.partial(_sc_body, nnz_pad),
        out_type=jax.ShapeDtypeStruct((NPB * OUT_F, L), jnp.float32),
        mesh=mesh,
        compiler_params=pltpu.CompilerParams(use_tc_tiling_on_sc=False),
        scratch_types=[
            pltpu.VMEM((nnz_pad,), jnp.int32),
            pltpu.VMEM((nnz_pad,), jnp.int32),
            pltpu.VMEM((nnz_pad,), jnp.float32),
            pltpu.VMEM((2, GB * K, L), jnp.float32),
            pltpu.VMEM((OUT_F, L), jnp.float32),
            pltpu.SemaphoreType.DMA((2,)),
        ],
    )(inputX, m0, m1, w)


def kernel(input, sparse_mask, weight, bias):
    assert input.shape == (BATCH, IN_F)
    nnz = sparse_mask.shape[0]
    blk = GB * K
    nnz_pad = ((nnz + blk - 1) // blk) * blk
    pad = nnz_pad - nnz

    inputX = (input.reshape(NPB, L, IN_F)
              .transpose(0, 2, 1)
              .reshape(NPB * IN_F, L))
    m0 = jnp.concatenate([sparse_mask[:, 0], jnp.zeros((pad,), jnp.int32)])
    m1 = jnp.concatenate([sparse_mask[:, 1], jnp.zeros((pad,), jnp.int32)])
    w = jnp.concatenate([weight, jnp.zeros((pad,), weight.dtype)])

    outX = _sc_call(inputX, m0, m1, w, nnz_pad=nnz_pad)
    out = (outX.reshape(NPB, OUT_F, L)
           .transpose(0, 2, 1)
           .reshape(BATCH, OUT_F))
    return out + bias[None, :]
